# baseline (speedup 1.0000x reference)
import functools

import jax
import jax.numpy as jnp
from jax import lax
from jax.experimental import pallas as pl
from jax.experimental.pallas import tpu as pltpu
from jax.experimental.pallas import tpu_sc as plsc

EPS = 1e-5

N = 10000
E = 320000
D = 128
NC = 2
NS = 16
NW = NC * NS
EPW = E // NW
CH = 80
NCH = EPW // CH
NV = D // 16
SB = 624
SREM = N - NS * SB

_mesh = plsc.VectorSubcoreMesh(
    core_axis_name="c", subcore_axis_name="s", num_cores=NC, num_subcores=NS)


def _tc_pre_body(x_ref, w_ref, y_ref):
    y_ref[...] = jnp.dot(x_ref[...], w_ref[...],
                         preferred_element_type=jnp.float32,
                    precision=lax.Precision.HIGHEST)


def _tc_pre(x, w):
    return pl.pallas_call(
        _tc_pre_body,
        out_shape=jax.ShapeDtypeStruct((N, D), jnp.float32),
    )(x, w)


NBUF = 5
PD = 3
NGRP = NCH // NBUF


@functools.partial(
    pl.kernel,
    out_type=(jax.ShapeDtypeStruct((E, D), jnp.float32),
              jax.ShapeDtypeStruct((NW, 2 * D), jnp.float32)),
    mesh=_mesh,
    scratch_types=[
        pltpu.VMEM((NCH, CH), jnp.int32),
        pltpu.VMEM((NCH, CH), jnp.int32),
        pltpu.VMEM((CH, D), jnp.float32),
        pltpu.VMEM((CH, D), jnp.float32),
        pltpu.VMEM((CH, D), jnp.float32),
        pltpu.VMEM((2 * D,), jnp.float32),
        pltpu.SemaphoreType.DMA,
        pltpu.SemaphoreType.DMA,
    ],
)
def _sc_pass1(y_hbm, row_hbm, col_hbm, h_hbm, stats_hbm,
              rowi_v, coli_v, yr_v, yc_v, hb_v, st_v, sem1, sem2):
    c = lax.axis_index("c")
    s_ = lax.axis_index("s")
    wid = s_ * NC + c
    ebase = wid * EPW

    pltpu.sync_copy(row_hbm.at[wid], rowi_v)
    pltpu.sync_copy(col_hbm.at[wid], coli_v)

    zero = jnp.zeros((16,), jnp.float32)
    init = tuple(zero for _ in range(2 * NV))

    def chunk_body(j, acc):
        cp1 = pltpu.async_copy(y_hbm.at[rowi_v.at[j]], yr_v, sem1)
        cp2 = pltpu.async_copy(y_hbm.at[coli_v.at[j]], yc_v, sem2)
        cp1.wait()
        cp2.wait()

        def edge_body(i, a):
            out = []
            for jj in range(NV):
                sl = pl.ds(jj * 16, 16)
                hh = yr_v[i, sl] - yc_v[i, sl]
                hb_v[i, sl] = hh
                out.append(a[jj] + hh)
                out.append(a[NV + jj] + hh * hh)
            return tuple(out[::2]) + tuple(out[1::2])

        acc = lax.fori_loop(0, CH, edge_body, acc)
        off = pl.multiple_of(ebase + j * CH, 8)
        pltpu.sync_copy(hb_v, h_hbm.at[pl.ds(off, CH)])
        return acc

    acc = lax.fori_loop(0, NCH, chunk_body, init)
    for jj in range(NV):
        st_v[pl.ds(jj * 16, 16)] = acc[jj]
        st_v[pl.ds(D + jj * 16, 16)] = acc[NV + jj]
    pltpu.sync_copy(st_v, stats_hbm.at[wid])


CH2 = 80
NCH2 = EPW // CH2


BIGC = 2048.0


@functools.partial(
    pl.kernel,
    out_type=jax.ShapeDtypeStruct((NC, N, D), jnp.float32),
    mesh=_mesh,
    scratch_types=[
        pltpu.VMEM((NCH2, CH2), jnp.int32),
        pltpu.VMEM((CH2, D), jnp.float32),
        pltpu.VMEM((2 * D,), jnp.float32),
        pltpu.VMEM_SHARED((N, D), jnp.float32),
        pltpu.SemaphoreType.DMA,
    ],
)
def _sc_pass2(h_hbm, row_hbm, st_hbm, zrow_hbm, agg_hbm,
              rowi_v, hb_v, st_v, acc_sh, sem):
    c = lax.axis_index("c")
    s_ = lax.axis_index("s")
    wid = s_ * NC + c
    ebase = wid * EPW

    soff = pl.multiple_of(s_ * SB, 8)
    pltpu.sync_copy(zrow_hbm.at[pl.ds(soff, SB)], acc_sh.at[pl.ds(soff, SB)])

    @pl.when(s_ == NS - 1)
    def _():
        pltpu.sync_copy(zrow_hbm.at[pl.ds(NS * SB, SREM)],
                        acc_sh.at[pl.ds(NS * SB, SREM)])

    pltpu.sync_copy(row_hbm.at[wid], rowi_v)
    pltpu.sync_copy(st_hbm, st_v)

    svec = [st_v[pl.ds(jj * 16, 16)] for jj in range(NV)]
    tvec = [st_v[pl.ds(D + jj * 16, 16)] for jj in range(NV)]
    lanes = lax.iota(jnp.int32, 16)
    bigv = jnp.where(lanes == 15, BIGC, 0.0)

    plsc.subcore_barrier()

    def chunk_body(j, _):
        off = pl.multiple_of(ebase + j * CH2, 8)
        pltpu.sync_copy(h_hbm.at[pl.ds(off, CH2)], hb_v)

        def edge_body(i, carry):
            for jj in range(NV):
                sl = pl.ds(jj * 16, 16)
                v = hb_v[i, sl] * svec[jj] + tvec[jj]
                v = jnp.maximum(v, 0.0)
                if jj == NV - 1:
                    v = v + bigv
                hb_v[i, sl] = v
            return carry

        lax.fori_loop(0, CH2, edge_body, 0)
        pltpu.sync_copy(hb_v, acc_sh.at[rowi_v.at[j]], add=True)
        return 0

    lax.fori_loop(0, NCH2, chunk_body, 0)
    plsc.subcore_barrier()

    pltpu.sync_copy(acc_sh.at[pl.ds(soff, SB)],
                    agg_hbm.at[c].at[pl.ds(soff, SB)])

    @pl.when(s_ == NS - 1)
    def _():
        pltpu.sync_copy(acc_sh.at[pl.ds(NS * SB, SREM)],
                        agg_hbm.at[c].at[pl.ds(NS * SB, SREM)])


def _tc_post_body(x_ref, agg_ref, w2e_ref, bb2e_ref,
                  w1a_ref, g1a_ref, b1a_ref, w2a_ref, bb2a_ref, out_ref):
    acc = agg_ref[0] + agg_ref[1]
    cnt = jnp.floor(acc[:, D - 1] * (1.0 / BIGC))
    is_last = lax.broadcasted_iota(jnp.int32, (1, D), 1) == D - 1
    agg_a = acc - jnp.where(is_last, (BIGC * cnt)[:, None], 0.0)
    agg = jnp.dot(agg_a, w2e_ref[...], preferred_element_type=jnp.float32,
                    precision=lax.Precision.HIGHEST)
    agg = agg + cnt[:, None] * bb2e_ref[...]
    z = (jnp.dot(x_ref[...], w1a_ref[0], preferred_element_type=jnp.float32,
                    precision=lax.Precision.HIGHEST)
         + jnp.dot(agg, w1a_ref[1], preferred_element_type=jnp.float32,
                    precision=lax.Precision.HIGHEST))
    mean = jnp.mean(z, axis=0)
    zc = z - mean
    var = jnp.mean(zc * zc, axis=0)
    zb = zc * lax.rsqrt(var + EPS) * g1a_ref[...] + b1a_ref[...]
    zb = jnp.maximum(zb, 0.0)
    out_ref[...] = (jnp.dot(zb, w2a_ref[...], preferred_element_type=jnp.float32,
                    precision=lax.Precision.HIGHEST)
                    + bb2a_ref[...])


def _tc_post(x, agg, W2e, bb2e, W1a, g1a, b1a, W2a, bb2a):
    return pl.pallas_call(
        _tc_post_body,
        out_shape=jax.ShapeDtypeStruct((N, D), jnp.float32),
    )(x, agg, W2e, bb2e, W1a.reshape(2, D, D), g1a, b1a, W2a, bb2a)


@jax.jit
def kernel(x, edge_index, W1e, g1e, b1e, W2e, bb2e, W1a, g1a, b1a, W2a, bb2a):
    row = edge_index[0].astype(jnp.int32)
    col = edge_index[1].astype(jnp.int32)
    row3 = row.reshape(NW, NCH, CH)
    col3 = col.reshape(NW, NCH, CH)

    y = _tc_pre(x, W1e)
    h, stats = _sc_pass1(y, row3, col3)

    sums = jnp.sum(stats[:, :D], axis=0)
    sqs = jnp.sum(stats[:, D:], axis=0)
    mean = sums / E
    var = sqs / E - mean * mean
    s = g1e * lax.rsqrt(var + EPS)
    t = b1e - mean * s
    st = jnp.concatenate([s, t])

    zrow = jnp.zeros((N, D), jnp.float32)
    row80 = row.reshape(NW, NCH2, CH2)
    agg_p = _sc_pass2(h, row80, st, zrow)

    return _tc_post(x, agg_p, W2e, bb2e, W1a, g1a, b1a, W2a, bb2a)


# speedup vs baseline: 1.2374x; 1.2374x over previous
import functools

import jax
import jax.numpy as jnp
from jax import lax
from jax.experimental import pallas as pl
from jax.experimental.pallas import tpu as pltpu
from jax.experimental.pallas import tpu_sc as plsc

EPS = 1e-5

N = 10000
E = 320000
D = 128
NC = 2
NS = 16
NW = NC * NS
EPW = E // NW
CH = 80
NCH = EPW // CH
NV = D // 16
SB = 624
SREM = N - NS * SB

_mesh = plsc.VectorSubcoreMesh(
    core_axis_name="c", subcore_axis_name="s", num_cores=NC, num_subcores=NS)


def _tc_pre_body(x_ref, w_ref, y_ref):
    y_ref[...] = jnp.dot(x_ref[...], w_ref[...],
                         preferred_element_type=jnp.float32,
                    precision=lax.Precision.HIGHEST)


def _tc_pre(x, w):
    return pl.pallas_call(
        _tc_pre_body,
        out_shape=jax.ShapeDtypeStruct((N, D), jnp.float32),
    )(x, w)


NBUF = 5
PD = 3
NGRP = NCH // NBUF


@functools.partial(
    pl.kernel,
    out_type=(jax.ShapeDtypeStruct((E, D), jnp.float32),
              jax.ShapeDtypeStruct((NW, 2 * D), jnp.float32)),
    mesh=_mesh,
    scratch_types=[
        pltpu.VMEM((NCH, CH), jnp.int32),
        pltpu.VMEM((NCH, CH), jnp.int32),
        pltpu.VMEM((2, CH, D), jnp.float32),
        pltpu.VMEM((2, CH, D), jnp.float32),
        pltpu.VMEM((2 * D,), jnp.float32),
        [pltpu.SemaphoreType.DMA] * 4,
        [pltpu.SemaphoreType.DMA] * 2,
    ],
)
def _sc_pass1(y_hbm, row_hbm, col_hbm, h_hbm, stats_hbm,
              rowi_v, coli_v, yrb, ycb, st_v, semg, sems):
    c = lax.axis_index("c")
    s_ = lax.axis_index("s")
    wid = s_ * NC + c
    ebase = wid * EPW

    pltpu.sync_copy(row_hbm.at[wid], rowi_v)
    pltpu.sync_copy(col_hbm.at[wid], coli_v)

    zero = jnp.zeros((16,), jnp.float32)
    init = tuple(zero for _ in range(2 * NV))

    def gathers(j, b):
        cp1 = pltpu.async_copy(y_hbm.at[rowi_v.at[j]], yrb.at[b], semg[2 * b])
        cp2 = pltpu.async_copy(y_hbm.at[coli_v.at[j]], ycb.at[b],
                               semg[2 * b + 1])
        return cp1, cp2

    def compute(b, acc):
        def edge_body(i, a):
            out = []
            for jj in range(NV):
                sl = pl.ds(jj * 16, 16)
                hh = yrb[b, i, sl] - ycb[b, i, sl]
                yrb[b, i, sl] = hh
                out.append(a[jj] + hh)
                out.append(a[NV + jj] + hh * hh)
            return tuple(out[::2]) + tuple(out[1::2])

        return lax.fori_loop(0, CH, edge_body, acc)

    def store(j, b):
        off = pl.multiple_of(ebase + j * CH, 8)
        return pltpu.async_copy(yrb.at[b], h_hbm.at[pl.ds(off, CH)], sems[b])

    def pair_body(p, acc):
        j0 = 2 * p
        a1, a2 = gathers(j0, 0)
        b1, b2 = gathers(j0 + 1, 1)
        a1.wait()
        a2.wait()
        acc = compute(0, acc)
        stA = store(j0, 0)
        b1.wait()
        b2.wait()
        acc = compute(1, acc)
        stB = store(j0 + 1, 1)
        stA.wait()
        stB.wait()
        return acc

    acc = lax.fori_loop(0, NCH // 2, pair_body, init)
    a1, a2 = gathers(NCH - 1, 0)
    a1.wait()
    a2.wait()
    acc = compute(0, acc)
    store(NCH - 1, 0).wait()
    for jj in range(NV):
        st_v[pl.ds(jj * 16, 16)] = acc[jj]
        st_v[pl.ds(D + jj * 16, 16)] = acc[NV + jj]
    pltpu.sync_copy(st_v, stats_hbm.at[wid])


CH2 = 80
NCH2 = EPW // CH2


BIGC = 2048.0


@functools.partial(
    pl.kernel,
    out_type=jax.ShapeDtypeStruct((NC, N, D), jnp.float32),
    mesh=_mesh,
    scratch_types=[
        pltpu.VMEM((NCH2, CH2), jnp.int32),
        pltpu.VMEM((2, CH2, D), jnp.float32),
        pltpu.VMEM((2 * D,), jnp.float32),
        pltpu.VMEM_SHARED((N, D), jnp.float32),
        [pltpu.SemaphoreType.DMA] * 2,
        [pltpu.SemaphoreType.DMA] * 2,
    ],
)
def _sc_pass2(h_hbm, row_hbm, st_hbm, zrow_hbm, agg_hbm,
              rowi_v, hbb, st_v, acc_sh, semld, semsc):
    c = lax.axis_index("c")
    s_ = lax.axis_index("s")
    wid = s_ * NC + c
    ebase = wid * EPW

    soff = pl.multiple_of(s_ * SB, 8)
    pltpu.sync_copy(zrow_hbm.at[pl.ds(soff, SB)], acc_sh.at[pl.ds(soff, SB)])

    @pl.when(s_ == NS - 1)
    def _():
        pltpu.sync_copy(zrow_hbm.at[pl.ds(NS * SB, SREM)],
                        acc_sh.at[pl.ds(NS * SB, SREM)])

    pltpu.sync_copy(row_hbm.at[wid], rowi_v)
    pltpu.sync_copy(st_hbm, st_v)

    svec = [st_v[pl.ds(jj * 16, 16)] for jj in range(NV)]
    tvec = [st_v[pl.ds(D + jj * 16, 16)] for jj in range(NV)]
    lanes = lax.iota(jnp.int32, 16)
    bigv = jnp.where(lanes == 15, BIGC, 0.0)

    plsc.subcore_barrier()

    def load(j, b):
        off = pl.multiple_of(ebase + j * CH2, 8)
        return pltpu.async_copy(h_hbm.at[pl.ds(off, CH2)], hbb.at[b],
                                semld[b])

    def compute(b):
        def edge_body(i, carry):
            for jj in range(NV):
                sl = pl.ds(jj * 16, 16)
                v = hbb[b, i, sl] * svec[jj] + tvec[jj]
                v = jnp.maximum(v, 0.0)
                if jj == NV - 1:
                    v = v + bigv
                hbb[b, i, sl] = v
            return carry

        lax.fori_loop(0, CH2, edge_body, 0)

    def scatter(j, b):
        return pltpu.async_copy(hbb.at[b], acc_sh.at[rowi_v.at[j]], semsc[b],
                                add=True)

    def pair_body(p, carry):
        j0 = 2 * p
        cpA = load(j0, 0)
        cpB = load(j0 + 1, 1)
        cpA.wait()
        compute(0)
        scA = scatter(j0, 0)
        cpB.wait()
        compute(1)
        scA.wait()
        scB = scatter(j0 + 1, 1)
        scB.wait()
        return carry

    lax.fori_loop(0, NCH2 // 2, pair_body, 0)
    load(NCH2 - 1, 0).wait()
    compute(0)
    scatter(NCH2 - 1, 0).wait()
    plsc.subcore_barrier()

    pltpu.sync_copy(acc_sh.at[pl.ds(soff, SB)],
                    agg_hbm.at[c].at[pl.ds(soff, SB)])

    @pl.when(s_ == NS - 1)
    def _():
        pltpu.sync_copy(acc_sh.at[pl.ds(NS * SB, SREM)],
                        agg_hbm.at[c].at[pl.ds(NS * SB, SREM)])


def _tc_post_body(x_ref, agg_ref, w2e_ref, bb2e_ref,
                  w1a_ref, g1a_ref, b1a_ref, w2a_ref, bb2a_ref, out_ref):
    acc = agg_ref[0] + agg_ref[1]
    cnt = jnp.floor(acc[:, D - 1] * (1.0 / BIGC))
    is_last = lax.broadcasted_iota(jnp.int32, (1, D), 1) == D - 1
    agg_a = acc - jnp.where(is_last, (BIGC * cnt)[:, None], 0.0)
    agg = jnp.dot(agg_a, w2e_ref[...], preferred_element_type=jnp.float32,
                    precision=lax.Precision.HIGHEST)
    agg = agg + cnt[:, None] * bb2e_ref[...]
    z = (jnp.dot(x_ref[...], w1a_ref[0], preferred_element_type=jnp.float32,
                    precision=lax.Precision.HIGHEST)
         + jnp.dot(agg, w1a_ref[1], preferred_element_type=jnp.float32,
                    precision=lax.Precision.HIGHEST))
    mean = jnp.mean(z, axis=0)
    zc = z - mean
    var = jnp.mean(zc * zc, axis=0)
    zb = zc * lax.rsqrt(var + EPS) * g1a_ref[...] + b1a_ref[...]
    zb = jnp.maximum(zb, 0.0)
    out_ref[...] = (jnp.dot(zb, w2a_ref[...], preferred_element_type=jnp.float32,
                    precision=lax.Precision.HIGHEST)
                    + bb2a_ref[...])


def _tc_post(x, agg, W2e, bb2e, W1a, g1a, b1a, W2a, bb2a):
    return pl.pallas_call(
        _tc_post_body,
        out_shape=jax.ShapeDtypeStruct((N, D), jnp.float32),
    )(x, agg, W2e, bb2e, W1a.reshape(2, D, D), g1a, b1a, W2a, bb2a)


@jax.jit
def kernel(x, edge_index, W1e, g1e, b1e, W2e, bb2e, W1a, g1a, b1a, W2a, bb2a):
    row = edge_index[0].astype(jnp.int32)
    col = edge_index[1].astype(jnp.int32)
    row3 = row.reshape(NW, NCH, CH)
    col3 = col.reshape(NW, NCH, CH)

    y = _tc_pre(x, W1e)
    h, stats = _sc_pass1(y, row3, col3)

    sums = jnp.sum(stats[:, :D], axis=0)
    sqs = jnp.sum(stats[:, D:], axis=0)
    mean = sums / E
    var = sqs / E - mean * mean
    s = g1e * lax.rsqrt(var + EPS)
    t = b1e - mean * s
    st = jnp.concatenate([s, t])

    zrow = jnp.zeros((N, D), jnp.float32)
    row80 = row.reshape(NW, NCH2, CH2)
    agg_p = _sc_pass2(h, row80, st, zrow)

    return _tc_post(x, agg_p, W2e, bb2e, W1a, g1a, b1a, W2a, bb2a)


# speedup vs baseline: 1.3688x; 1.1062x over previous
import functools

import jax
import jax.numpy as jnp
from jax import lax
from jax.experimental import pallas as pl
from jax.experimental.pallas import tpu as pltpu
from jax.experimental.pallas import tpu_sc as plsc

EPS = 1e-5

N = 10000
E = 320000
D = 128
NC = 2
NS = 16
NW = NC * NS
EPW = E // NW
CH = 80
NCH = EPW // CH
NV = D // 16
SB = 624
SREM = N - NS * SB

_mesh = plsc.VectorSubcoreMesh(
    core_axis_name="c", subcore_axis_name="s", num_cores=NC, num_subcores=NS)


def _tc_pre_body(x_ref, w_ref, y_ref):
    y_ref[...] = jnp.dot(x_ref[...], w_ref[...],
                         preferred_element_type=jnp.float32,
                    precision=lax.Precision.HIGHEST)


def _tc_pre(x, w):
    return pl.pallas_call(
        _tc_pre_body,
        out_shape=jax.ShapeDtypeStruct((N, D), jnp.float32),
    )(x, w)


NBUF = 5
PD = 3
NGRP = NCH // NBUF


@functools.partial(
    pl.kernel,
    out_type=(jax.ShapeDtypeStruct((E, D), jnp.float32),
              jax.ShapeDtypeStruct((NW, 2 * D), jnp.float32)),
    mesh=_mesh,
    scratch_types=[
        pltpu.VMEM((NCH, CH), jnp.int32),
        pltpu.VMEM((NCH, CH), jnp.int32),
        pltpu.VMEM((4, CH, D), jnp.float32),
        pltpu.VMEM((4, CH, D), jnp.float32),
        pltpu.VMEM((2 * D,), jnp.float32),
        [pltpu.SemaphoreType.DMA] * 8,
        [pltpu.SemaphoreType.DMA] * 4,
    ],
)
def _sc_pass1(y_hbm, row_hbm, col_hbm, h_hbm, stats_hbm,
              rowi_v, coli_v, yrb, ycb, st_v, semg, sems):
    c = lax.axis_index("c")
    s_ = lax.axis_index("s")
    wid = s_ * NC + c
    ebase = wid * EPW

    pltpu.sync_copy(row_hbm.at[wid], rowi_v)
    pltpu.sync_copy(col_hbm.at[wid], coli_v)

    zero = jnp.zeros((16,), jnp.float32)
    init = tuple(zero for _ in range(2 * NV))

    def gathers(j, b):
        cp1 = pltpu.async_copy(y_hbm.at[rowi_v.at[j]], yrb.at[b], semg[2 * b])
        cp2 = pltpu.async_copy(y_hbm.at[coli_v.at[j]], ycb.at[b],
                               semg[2 * b + 1])
        return cp1, cp2

    def compute(b, acc):
        def edge_body(i, a):
            out = []
            for jj in range(NV):
                sl = pl.ds(jj * 16, 16)
                hh = yrb[b, i, sl] - ycb[b, i, sl]
                yrb[b, i, sl] = hh
                out.append(a[jj] + hh)
                out.append(a[NV + jj] + hh * hh)
            return tuple(out[::2]) + tuple(out[1::2])

        return lax.fori_loop(0, CH, edge_body, acc)

    def store(j, b):
        off = pl.multiple_of(ebase + j * CH, 8)
        return pltpu.async_copy(yrb.at[b], h_hbm.at[pl.ds(off, CH)], sems[b])

    QB = 4

    def quad_body(p, acc):
        j0 = QB * p
        cps = [gathers(j0 + b, b) for b in range(QB)]
        sts = []
        for b in range(QB):
            cps[b][0].wait()
            cps[b][1].wait()
            acc = compute(b, acc)
            sts.append(store(j0 + b, b))
        for st in sts:
            st.wait()
        return acc

    acc = lax.fori_loop(0, NCH // QB, quad_body, init)
    a1, a2 = gathers(NCH - 1, 0)
    a1.wait()
    a2.wait()
    acc = compute(0, acc)
    store(NCH - 1, 0).wait()
    for jj in range(NV):
        st_v[pl.ds(jj * 16, 16)] = acc[jj]
        st_v[pl.ds(D + jj * 16, 16)] = acc[NV + jj]
    pltpu.sync_copy(st_v, stats_hbm.at[wid])


CH2 = 80
NCH2 = EPW // CH2


BIGC = 2048.0


@functools.partial(
    pl.kernel,
    out_type=jax.ShapeDtypeStruct((NC, N, D), jnp.float32),
    mesh=_mesh,
    scratch_types=[
        pltpu.VMEM((NCH2, CH2), jnp.int32),
        pltpu.VMEM((2, CH2, D), jnp.float32),
        pltpu.VMEM((2 * D,), jnp.float32),
        pltpu.VMEM_SHARED((N, D), jnp.float32),
        [pltpu.SemaphoreType.DMA] * 2,
        [pltpu.SemaphoreType.DMA] * 2,
    ],
)
def _sc_pass2(h_hbm, row_hbm, st_hbm, zrow_hbm, agg_hbm,
              rowi_v, hbb, st_v, acc_sh, semld, semsc):
    c = lax.axis_index("c")
    s_ = lax.axis_index("s")
    wid = s_ * NC + c
    ebase = wid * EPW

    soff = pl.multiple_of(s_ * SB, 8)
    pltpu.sync_copy(zrow_hbm.at[pl.ds(soff, SB)], acc_sh.at[pl.ds(soff, SB)])

    @pl.when(s_ == NS - 1)
    def _():
        pltpu.sync_copy(zrow_hbm.at[pl.ds(NS * SB, SREM)],
                        acc_sh.at[pl.ds(NS * SB, SREM)])

    pltpu.sync_copy(row_hbm.at[wid], rowi_v)
    pltpu.sync_copy(st_hbm, st_v)

    svec = [st_v[pl.ds(jj * 16, 16)] for jj in range(NV)]
    tvec = [st_v[pl.ds(D + jj * 16, 16)] for jj in range(NV)]
    lanes = lax.iota(jnp.int32, 16)
    bigv = jnp.where(lanes == 15, BIGC, 0.0)

    plsc.subcore_barrier()

    def load(j, b):
        off = pl.multiple_of(ebase + j * CH2, 8)
        return pltpu.async_copy(h_hbm.at[pl.ds(off, CH2)], hbb.at[b],
                                semld[b])

    def compute(b):
        def edge_body(i, carry):
            for jj in range(NV):
                sl = pl.ds(jj * 16, 16)
                v = hbb[b, i, sl] * svec[jj] + tvec[jj]
                v = jnp.maximum(v, 0.0)
                if jj == NV - 1:
                    v = v + bigv
                hbb[b, i, sl] = v
            return carry

        lax.fori_loop(0, CH2, edge_body, 0)

    def scatter(j, b):
        return pltpu.async_copy(hbb.at[b], acc_sh.at[rowi_v.at[j]], semsc[b],
                                add=True)

    def pair_body(p, carry):
        j0 = 2 * p
        cpA = load(j0, 0)
        cpB = load(j0 + 1, 1)
        cpA.wait()
        compute(0)
        scA = scatter(j0, 0)
        cpB.wait()
        compute(1)
        scA.wait()
        scB = scatter(j0 + 1, 1)
        scB.wait()
        return carry

    lax.fori_loop(0, NCH2 // 2, pair_body, 0)
    load(NCH2 - 1, 0).wait()
    compute(0)
    scatter(NCH2 - 1, 0).wait()
    plsc.subcore_barrier()

    pltpu.sync_copy(acc_sh.at[pl.ds(soff, SB)],
                    agg_hbm.at[c].at[pl.ds(soff, SB)])

    @pl.when(s_ == NS - 1)
    def _():
        pltpu.sync_copy(acc_sh.at[pl.ds(NS * SB, SREM)],
                        agg_hbm.at[c].at[pl.ds(NS * SB, SREM)])


def _tc_post_body(x_ref, agg_ref, w2e_ref, bb2e_ref,
                  w1a_ref, g1a_ref, b1a_ref, w2a_ref, bb2a_ref, out_ref):
    acc = agg_ref[0] + agg_ref[1]
    cnt = jnp.floor(acc[:, D - 1] * (1.0 / BIGC))
    is_last = lax.broadcasted_iota(jnp.int32, (1, D), 1) == D - 1
    agg_a = acc - jnp.where(is_last, (BIGC * cnt)[:, None], 0.0)
    agg = jnp.dot(agg_a, w2e_ref[...], preferred_element_type=jnp.float32,
                    precision=lax.Precision.HIGHEST)
    agg = agg + cnt[:, None] * bb2e_ref[...]
    z = (jnp.dot(x_ref[...], w1a_ref[0], preferred_element_type=jnp.float32,
                    precision=lax.Precision.HIGHEST)
         + jnp.dot(agg, w1a_ref[1], preferred_element_type=jnp.float32,
                    precision=lax.Precision.HIGHEST))
    mean = jnp.mean(z, axis=0)
    zc = z - mean
    var = jnp.mean(zc * zc, axis=0)
    zb = zc * lax.rsqrt(var + EPS) * g1a_ref[...] + b1a_ref[...]
    zb = jnp.maximum(zb, 0.0)
    out_ref[...] = (jnp.dot(zb, w2a_ref[...], preferred_element_type=jnp.float32,
                    precision=lax.Precision.HIGHEST)
                    + bb2a_ref[...])


def _tc_post(x, agg, W2e, bb2e, W1a, g1a, b1a, W2a, bb2a):
    return pl.pallas_call(
        _tc_post_body,
        out_shape=jax.ShapeDtypeStruct((N, D), jnp.float32),
    )(x, agg, W2e, bb2e, W1a.reshape(2, D, D), g1a, b1a, W2a, bb2a)


@jax.jit
def kernel(x, edge_index, W1e, g1e, b1e, W2e, bb2e, W1a, g1a, b1a, W2a, bb2a):
    row = edge_index[0].astype(jnp.int32)
    col = edge_index[1].astype(jnp.int32)
    row3 = row.reshape(NW, NCH, CH)
    col3 = col.reshape(NW, NCH, CH)

    y = _tc_pre(x, W1e)
    h, stats = _sc_pass1(y, row3, col3)

    sums = jnp.sum(stats[:, :D], axis=0)
    sqs = jnp.sum(stats[:, D:], axis=0)
    mean = sums / E
    var = sqs / E - mean * mean
    s = g1e * lax.rsqrt(var + EPS)
    t = b1e - mean * s
    st = jnp.concatenate([s, t])

    zrow = jnp.zeros((N, D), jnp.float32)
    row80 = row.reshape(NW, NCH2, CH2)
    agg_p = _sc_pass2(h, row80, st, zrow)

    return _tc_post(x, agg_p, W2e, bb2e, W1a, g1a, b1a, W2a, bb2a)


# speedup vs baseline: 1.4174x; 1.0355x over previous
import functools

import jax
import jax.numpy as jnp
from jax import lax
from jax.experimental import pallas as pl
from jax.experimental.pallas import tpu as pltpu
from jax.experimental.pallas import tpu_sc as plsc

EPS = 1e-5

N = 10000
E = 320000
D = 128
NC = 2
NS = 16
NW = NC * NS
EPW = E // NW
CH = 80
NCH = EPW // CH
NV = D // 16
SB = 624
SREM = N - NS * SB

_mesh = plsc.VectorSubcoreMesh(
    core_axis_name="c", subcore_axis_name="s", num_cores=NC, num_subcores=NS)


def _tc_pre_body(x_ref, w_ref, y_ref):
    y_ref[...] = jnp.dot(x_ref[...], w_ref[...],
                         preferred_element_type=jnp.float32,
                    precision=lax.Precision.HIGHEST)


def _tc_pre(x, w):
    return pl.pallas_call(
        _tc_pre_body,
        out_shape=jax.ShapeDtypeStruct((N, D), jnp.float32),
    )(x, w)


NBUF = 5
PD = 3
NGRP = NCH // NBUF


@functools.partial(
    pl.kernel,
    out_type=(jax.ShapeDtypeStruct((E, D), jnp.float32),
              jax.ShapeDtypeStruct((NW, 2 * D), jnp.float32)),
    mesh=_mesh,
    scratch_types=[
        pltpu.VMEM((NCH, CH), jnp.int32),
        pltpu.VMEM((NCH, CH), jnp.int32),
        pltpu.VMEM((4, CH, D), jnp.float32),
        pltpu.VMEM((4, CH, D), jnp.float32),
        pltpu.VMEM((2 * D,), jnp.float32),
        [pltpu.SemaphoreType.DMA] * 8,
        [pltpu.SemaphoreType.DMA] * 4,
    ],
)
def _sc_pass1(y_hbm, row_hbm, col_hbm, h_hbm, stats_hbm,
              rowi_v, coli_v, yrb, ycb, st_v, semg, sems):
    c = lax.axis_index("c")
    s_ = lax.axis_index("s")
    wid = s_ * NC + c
    ebase = wid * EPW

    pltpu.sync_copy(row_hbm.at[wid], rowi_v)
    pltpu.sync_copy(col_hbm.at[wid], coli_v)

    zero = jnp.zeros((16,), jnp.float32)
    init = tuple(zero for _ in range(2 * NV))

    def gathers(j, b):
        cp1 = pltpu.async_copy(y_hbm.at[rowi_v.at[j]], yrb.at[b], semg[2 * b])
        cp2 = pltpu.async_copy(y_hbm.at[coli_v.at[j]], ycb.at[b],
                               semg[2 * b + 1])
        return cp1, cp2

    def compute(b, acc):
        def edge_body(i, a):
            out = []
            for jj in range(NV):
                sl = pl.ds(jj * 16, 16)
                hh = yrb[b, i, sl] - ycb[b, i, sl]
                yrb[b, i, sl] = hh
                out.append(a[jj] + hh)
                out.append(a[NV + jj] + hh * hh)
            return tuple(out[::2]) + tuple(out[1::2])

        return lax.fori_loop(0, CH, edge_body, acc)

    def store(j, b):
        off = pl.multiple_of(ebase + j * CH, 8)
        return pltpu.async_copy(yrb.at[b], h_hbm.at[pl.ds(off, CH)], sems[b])

    QB = 4

    def quad_body(p, acc):
        j0 = QB * p
        cps = [gathers(j0 + b, b) for b in range(QB)]
        sts = []
        for b in range(QB):
            cps[b][0].wait()
            cps[b][1].wait()
            acc = compute(b, acc)
            sts.append(store(j0 + b, b))
        for st in sts:
            st.wait()
        return acc

    acc = lax.fori_loop(0, NCH // QB, quad_body, init)
    a1, a2 = gathers(NCH - 1, 0)
    a1.wait()
    a2.wait()
    acc = compute(0, acc)
    store(NCH - 1, 0).wait()
    for jj in range(NV):
        st_v[pl.ds(jj * 16, 16)] = acc[jj]
        st_v[pl.ds(D + jj * 16, 16)] = acc[NV + jj]
    pltpu.sync_copy(st_v, stats_hbm.at[wid])


CH2 = 80
NCH2 = EPW // CH2


BIGC = 2048.0


@functools.partial(
    pl.kernel,
    out_type=jax.ShapeDtypeStruct((NC, N, D), jnp.float32),
    mesh=_mesh,
    scratch_types=[
        pltpu.VMEM((NCH2, CH2), jnp.int32),
        pltpu.VMEM((3, CH2, D), jnp.float32),
        pltpu.VMEM((2 * D,), jnp.float32),
        pltpu.VMEM_SHARED((N, D), jnp.float32),
        [pltpu.SemaphoreType.DMA] * 3,
        [pltpu.SemaphoreType.DMA] * 3,
    ],
)
def _sc_pass2(h_hbm, row_hbm, st_hbm, zrow_hbm, agg_hbm,
              rowi_v, hbb, st_v, acc_sh, semld, semsc):
    c = lax.axis_index("c")
    s_ = lax.axis_index("s")
    wid = s_ * NC + c
    ebase = wid * EPW

    soff = pl.multiple_of(s_ * SB, 8)
    pltpu.sync_copy(zrow_hbm.at[pl.ds(soff, SB)], acc_sh.at[pl.ds(soff, SB)])

    @pl.when(s_ == NS - 1)
    def _():
        pltpu.sync_copy(zrow_hbm.at[pl.ds(NS * SB, SREM)],
                        acc_sh.at[pl.ds(NS * SB, SREM)])

    pltpu.sync_copy(row_hbm.at[wid], rowi_v)
    pltpu.sync_copy(st_hbm, st_v)

    svec = [st_v[pl.ds(jj * 16, 16)] for jj in range(NV)]
    tvec = [st_v[pl.ds(D + jj * 16, 16)] for jj in range(NV)]
    lanes = lax.iota(jnp.int32, 16)
    bigv = jnp.where(lanes == 15, BIGC, 0.0)

    plsc.subcore_barrier()

    def load(j, b):
        off = pl.multiple_of(ebase + j * CH2, 8)
        return pltpu.async_copy(h_hbm.at[pl.ds(off, CH2)], hbb.at[b],
                                semld[b])

    def compute(b):
        def edge_body(i, carry):
            for jj in range(NV):
                sl = pl.ds(jj * 16, 16)
                v = hbb[b, i, sl] * svec[jj] + tvec[jj]
                v = jnp.maximum(v, 0.0)
                if jj == NV - 1:
                    v = v + bigv
                hbb[b, i, sl] = v
            return carry

        lax.fori_loop(0, CH2, edge_body, 0)

    def scatter(j, b):
        return pltpu.async_copy(hbb.at[b], acc_sh.at[rowi_v.at[j]], semsc[b],
                                add=True)

    def tri_body(p, carry):
        j0 = 3 * p
        cps = [load(j0 + b, b) for b in range(3)]
        cps[0].wait()
        compute(0)
        sc0 = scatter(j0, 0)
        cps[1].wait()
        compute(1)
        sc0.wait()
        sc1 = scatter(j0 + 1, 1)
        cps[2].wait()
        compute(2)
        sc1.wait()
        sc2 = scatter(j0 + 2, 2)
        sc2.wait()
        return carry

    lax.fori_loop(0, NCH2 // 3, tri_body, 0)
    for j in (NCH2 - 2, NCH2 - 1):
        load(j, 0).wait()
        compute(0)
        scatter(j, 0).wait()
    plsc.subcore_barrier()

    pltpu.sync_copy(acc_sh.at[pl.ds(soff, SB)],
                    agg_hbm.at[c].at[pl.ds(soff, SB)])

    @pl.when(s_ == NS - 1)
    def _():
        pltpu.sync_copy(acc_sh.at[pl.ds(NS * SB, SREM)],
                        agg_hbm.at[c].at[pl.ds(NS * SB, SREM)])


def _tc_post_body(x_ref, agg_ref, w2e_ref, bb2e_ref,
                  w1a_ref, g1a_ref, b1a_ref, w2a_ref, bb2a_ref, out_ref):
    acc = agg_ref[0] + agg_ref[1]
    cnt = jnp.floor(acc[:, D - 1] * (1.0 / BIGC))
    is_last = lax.broadcasted_iota(jnp.int32, (1, D), 1) == D - 1
    agg_a = acc - jnp.where(is_last, (BIGC * cnt)[:, None], 0.0)
    agg = jnp.dot(agg_a, w2e_ref[...], preferred_element_type=jnp.float32,
                    precision=lax.Precision.HIGHEST)
    agg = agg + cnt[:, None] * bb2e_ref[...]
    z = (jnp.dot(x_ref[...], w1a_ref[0], preferred_element_type=jnp.float32,
                    precision=lax.Precision.HIGHEST)
         + jnp.dot(agg, w1a_ref[1], preferred_element_type=jnp.float32,
                    precision=lax.Precision.HIGHEST))
    mean = jnp.mean(z, axis=0)
    zc = z - mean
    var = jnp.mean(zc * zc, axis=0)
    zb = zc * lax.rsqrt(var + EPS) * g1a_ref[...] + b1a_ref[...]
    zb = jnp.maximum(zb, 0.0)
    out_ref[...] = (jnp.dot(zb, w2a_ref[...], preferred_element_type=jnp.float32,
                    precision=lax.Precision.HIGHEST)
                    + bb2a_ref[...])


def _tc_post(x, agg, W2e, bb2e, W1a, g1a, b1a, W2a, bb2a):
    return pl.pallas_call(
        _tc_post_body,
        out_shape=jax.ShapeDtypeStruct((N, D), jnp.float32),
    )(x, agg, W2e, bb2e, W1a.reshape(2, D, D), g1a, b1a, W2a, bb2a)


@jax.jit
def kernel(x, edge_index, W1e, g1e, b1e, W2e, bb2e, W1a, g1a, b1a, W2a, bb2a):
    row = edge_index[0].astype(jnp.int32)
    col = edge_index[1].astype(jnp.int32)
    row3 = row.reshape(NW, NCH, CH)
    col3 = col.reshape(NW, NCH, CH)

    y = _tc_pre(x, W1e)
    h, stats = _sc_pass1(y, row3, col3)

    sums = jnp.sum(stats[:, :D], axis=0)
    sqs = jnp.sum(stats[:, D:], axis=0)
    mean = sums / E
    var = sqs / E - mean * mean
    s = g1e * lax.rsqrt(var + EPS)
    t = b1e - mean * s
    st = jnp.concatenate([s, t])

    zrow = jnp.zeros((N, D), jnp.float32)
    row80 = row.reshape(NW, NCH2, CH2)
    agg_p = _sc_pass2(h, row80, st, zrow)

    return _tc_post(x, agg_p, W2e, bb2e, W1a, g1a, b1a, W2a, bb2a)
